# baseline (speedup 1.0000x reference)
import functools

import jax
import jax.numpy as jnp
from jax import lax
from jax.experimental import pallas as pl
from jax.experimental.pallas import tpu as pltpu
from jax.experimental.pallas import tpu_sc as plsc

NG = 256
D = 128
NT = 3
R = 128
CW = 128


def _sc_segment_sums(n):
  info = plsc.get_sparse_core_info()
  nc, ns = info.num_cores, info.num_subcores
  nw = nc * ns
  nfull = n // R
  rem = n - nfull * R
  rows_per_tile = NG // ns

  mesh = plsc.VectorSubcoreMesh(core_axis_name="c", subcore_axis_name="s")

  out_type = [
      jax.ShapeDtypeStruct((nc, NT, NG, D), jnp.float32),
      jax.ShapeDtypeStruct((nc, NT, NG, CW), jnp.float32),
  ]
  scratch = [
      pltpu.VMEM((R,), jnp.int32),
      pltpu.VMEM((R,), jnp.int32),
      pltpu.VMEM((R, D), jnp.float32),
      pltpu.VMEM((R, D), jnp.float32),
      pltpu.VMEM((R, CW), jnp.float32),
      pltpu.VMEM((rem, ), jnp.int32) if rem else pltpu.VMEM((8,), jnp.int32),
      pltpu.VMEM((max(rem, 1), D), jnp.float32),
      pltpu.SemaphoreType.DMA,
      pltpu.SemaphoreType.DMA,
      pltpu.SemaphoreType.DMA,
      pltpu.SemaphoreType.DMA,
      pltpu.VMEM((16, D), jnp.float32),
      pltpu.VMEM((16, D), jnp.float32),
      pltpu.VMEM((16, CW), jnp.float32),
      pltpu.VMEM((16, D), jnp.float32),
      pltpu.VMEM((16, D), jnp.float32),
      pltpu.VMEM((16, CW), jnp.float32),
      pltpu.VMEM((16, CW), jnp.float32),
      pltpu.SMEM((2,), jnp.int32),
  ] + [pltpu.VMEM_SHARED((NG, D), jnp.float32) for _ in range(NT)] \
    + [pltpu.VMEM_SHARED((NG, CW), jnp.float32) for _ in range(NT)]

  @functools.partial(pl.kernel, mesh=mesh, out_type=out_type,
                     scratch_types=scratch)
  def k(h0, s0, h1, s1, h2, s2, ones_hbm, zacc_hbm, zcnt_hbm,
        acc_out, cnt_out,
        seg_v0, seg_v1, rows_v0, rows_v1, ones_v, segr_v, rowsr_v,
        sem0, sem1, ssem0, ssem1, sums_v0, sums_v1, cntr_v,
        sums2_v0, sums2_v1, cnt2_v0, cnt2_v1, path_sm,
        acc0_sh, acc1_sh, acc2_sh, cnt0_sh, cnt1_sh, cnt2_sh):
    accs = (acc0_sh, acc1_sh, acc2_sh)
    cnts = (cnt0_sh, cnt1_sh, cnt2_sh)
    bufs = ((seg_v0, rows_v0, sem0), (seg_v1, rows_v1, sem1))
    ssems = (ssem0, ssem1)
    sumsb = (sums_v0, sums_v1)
    sums2b = (sums2_v0, sums2_v1)
    cnt2b = (cnt2_v0, cnt2_v1)
    c = lax.axis_index("c")
    s = lax.axis_index("s")
    w = s * nc + c

    zsl = pl.ds(s * rows_per_tile, rows_per_tile)
    for t in range(NT):
      pltpu.sync_copy(zacc_hbm.at[zsl], accs[t].at[zsl])
      pltpu.sync_copy(zcnt_hbm.at[zsl], cnts[t].at[zsl])
    pltpu.sync_copy(ones_hbm, ones_v)
    for buf in (sums_v0, sums_v1, sums2_v0, sums2_v1):
      pltpu.sync_copy(zacc_hbm.at[pl.ds(0, 16)], buf)
    for buf in (cntr_v, cnt2_v0, cnt2_v1):
      pltpu.sync_copy(zcnt_hbm.at[pl.ds(0, 16)], buf)
    path_sm[0] = 0
    path_sm[1] = 0
    for j in range(CW // 16):
      cntr_v[0, pl.ds(16 * j, 16)] = jnp.full((16,), 1.0, jnp.float32)
    plsc.subcore_barrier()

    nk = (nfull - w + nw - 1) // nw

    def issue(i, segb, rowsb, sem, seg, h):
      base = (w + i * nw) * R
      pltpu.async_copy(seg.at[pl.ds(base, R)], segb, sem)
      pltpu.async_copy(h.at[pl.ds(base, R)], rowsb, sem)

    def drain(segb, rowsb, sem, seg, h):
      pltpu.make_async_copy(seg.at[pl.ds(0, R)], segb, sem).wait()
      pltpu.make_async_copy(h.at[pl.ds(0, R)], rowsb, sem).wait()

    def wait_small(q):
      pltpu.make_async_copy(zacc_hbm.at[pl.ds(0, 16)], sumsb[q],
                            ssems[q]).wait()
      pltpu.make_async_copy(zcnt_hbm.at[pl.ds(0, 16)], cntr_v,
                            ssems[q]).wait()

    def wait_big(q, h):
      pltpu.make_async_copy(h.at[pl.ds(0, R)], bufs[q][1], ssems[q]).wait()
      pltpu.make_async_copy(ones_hbm, ones_v, ssems[q]).wait()

    def drain_outstanding(q, h):
      pq = path_sm[q]
      @pl.when(pq == 1)
      def _():
        wait_small(q)
      @pl.when(pq == 2)
      def _():
        wait_big(q, h)
      path_sm[q] = 0

    for t, (h, seg) in enumerate(((h0, s0), (h1, s1), (h2, s2))):
      issue(0, *bufs[0], seg, h)

      def body(i, carry, h=h, seg=seg, t=t):
        for p in range(2):
          @pl.when(lax.rem(i, 2) == p)
          def _(p=p):
            segb, rowsb, sem = bufs[p]
            sums_v = sumsb[p]
            sums2_v = sums2b[p]
            cnt2_v = cnt2b[p]
            ssem = ssems[p]
            drain(segb, rowsb, sem, seg, h)
            drain_outstanding(1 - p, h)
            @pl.when(i + 1 < nk)
            def _():
              issue(i + 1, *bufs[1 - p], seg, h)
            v0 = segb[pl.ds(0, 16)]
            vlast = segb[pl.ds(R - 16, 16)]
            first = v0[0]
            last = vlast[15]
            uni = first == last

            @pl.when(uni)
            def _():
              def sbody(r, acc):
                out = []
                for j in range(D // 16):
                  a = acc[j]
                  for u in range(8):
                    a = a + rowsb[8 * r + u, pl.ds(16 * j, 16)]
                  out.append(a)
                return tuple(out)
              acc = lax.fori_loop(
                  0, R // 8, sbody,
                  tuple(jnp.zeros((16,), jnp.float32)
                        for _ in range(D // 16)))
              for j in range(D // 16):
                sums_v[0, pl.ds(16 * j, 16)] = acc[j]
              pltpu.async_copy(sums_v, accs[t].at[v0], ssem, add=True)
              pltpu.async_copy(cntr_v, cnts[t].at[v0], ssem, add=True)
              path_sm[p] = 1

            @pl.when(jnp.logical_not(uni))
            def _():
              svs = [segb[pl.ds(16 * g, 16)] for g in range(R // 16)]
              e0 = [sv[0] for sv in svs]
              e15 = [sv[15] for sv in svs]
              m = [a != z for a, z in zip(e0, e15)]
              nnu = jnp.int32(0)
              gstar = jnp.int32(0)
              unif_ok = jnp.bool_(True)
              for g in range(R // 16):
                nnu = nnu + m[g].astype(jnp.int32)
                gstar = gstar + jnp.logical_and(
                    e0[g] == first, e15[g] == first).astype(jnp.int32)
                unif_ok = jnp.logical_and(
                    unif_ok,
                    m[g] | (e0[g] == first) | (e0[g] == last))
              svb = svs[-1]
              for g in range(R // 16 - 2, -1, -1):
                svb = jnp.where(m[g], svs[g], svb)
              minv = svs[0]
              maxv = svs[0]
              for g in range(1, R // 16):
                minv = jnp.minimum(minv, svs[g])
                maxv = jnp.maximum(maxv, svs[g])
              nfv = jnp.zeros((16,), jnp.float32)
              nlv = jnp.zeros((16,), jnp.float32)
              for g in range(R // 16):
                nfv = nfv + jnp.where(svs[g] == minv, 1.0, 0.0)
                nlv = nlv + jnp.where(svs[g] == maxv, 1.0, 0.0)
              b_within = jnp.int32(0)
              inset = jnp.bool_(True)
              for kk in range(16):
                ev = svb[kk]
                b_within = b_within + (ev == first).astype(jnp.int32)
                inset = jnp.logical_and(inset, (ev == first) | (ev == last))
              b = 16 * gstar + b_within
              two = ((nnu <= 1) & inset & unif_ok
                     & jnp.logical_not(m[0]) & jnp.logical_not(m[-1]))

              @pl.when(two)
              def _():
                def rbody(r, acc):
                  return tuple(acc[j] + rowsb[r, pl.ds(16 * j, 16)]
                               for j in range(D // 16))
                z8 = tuple(jnp.zeros((16,), jnp.float32)
                           for _ in range(D // 16))
                s1 = lax.fori_loop(0, b, rbody, z8)
                s2 = lax.fori_loop(b, R, rbody, z8)
                for j in range(D // 16):
                  sums2_v[0, pl.ds(16 * j, 16)] = s1[j]
                  sums2_v[1, pl.ds(16 * j, 16)] = s2[j]
                cnt2_v[0, pl.ds(0, 16)] = nfv
                cnt2_v[1, pl.ds(0, 16)] = nlv
                lane1 = lax.iota(jnp.int32, 16) == 1
                idx2 = jnp.where(lane1, maxv, minv)
                pltpu.async_copy(sums2_v, accs[t].at[idx2], ssem, add=True)
                pltpu.async_copy(cnt2_v, cnts[t].at[idx2], ssem, add=True)
                path_sm[p] = 1

              @pl.when(jnp.logical_not(two))
              def _():
                pltpu.async_copy(rowsb, accs[t].at[segb], ssem, add=True)
                pltpu.async_copy(ones_v, cnts[t].at[segb], ssem, add=True)
                path_sm[p] = 2
        return carry
      lax.fori_loop(0, nk, body, 0)
      drain_outstanding(0, h)
      drain_outstanding(1, h)

    if rem:
      @pl.when(w == nw - 1)
      def _():
        for t, (h, seg) in enumerate(((h0, s0), (h1, s1), (h2, s2))):
          pltpu.sync_copy(seg.at[pl.ds(nfull * R, rem)], segr_v)
          pltpu.sync_copy(h.at[pl.ds(nfull * R, rem)], rowsr_v)
          pltpu.sync_copy(rowsr_v, accs[t].at[segr_v], add=True)
          pltpu.sync_copy(ones_v.at[pl.ds(0, rem)], cnts[t].at[segr_v],
                          add=True)

    plsc.subcore_barrier()

    for t in range(NT):
      pltpu.sync_copy(accs[t].at[zsl], acc_out.at[c, t, zsl])
      pltpu.sync_copy(cnts[t].at[zsl], cnt_out.at[c, t, zsl])

  return k


def _tc_segment_sums(n_tc):
  nb = n_tc // R

  def body(s0_ref, s1_ref, s2_ref, h0_ref, h1_ref, h2_ref,
           acc_out, cnt_out, acc3, cnt3):
    i = pl.program_id(0)

    @pl.when(i == 0)
    def _():
      acc3[...] = jnp.zeros_like(acc3)
      cnt3[...] = jnp.zeros_like(cnt3)

    riota = lax.broadcasted_iota(jnp.int32, (R, D), 0)
    for t, (sref, href) in enumerate(
        ((s0_ref, h0_ref), (s1_ref, h1_ref), (s2_ref, h2_ref))):
      a = sref[0]
      z = sref[R - 1]
      rows = href[...]

      @pl.when(a == z)
      def _(t=t, a=a, rows=rows):
        tot = jnp.sum(rows, axis=0, keepdims=True)
        acc3[t, pl.ds(a, 1), :] += tot
        cnt3[t, pl.ds(a, 1), 0:1] += jnp.float32(R)

      @pl.when(a != z)
      def _(t=t, a=a, z=z, rows=rows, sref=sref):
        def cond(cr):
          return cr[0] < cr[1]
        def step(cr):
          lo, hi = cr
          mid = (lo + hi) // 2
          eq = sref[mid] == a
          return (jnp.where(eq, mid + 1, lo), jnp.where(eq, hi, mid))
        b = lax.while_loop(cond, step, (jnp.int32(1), jnp.int32(R)))[0]
        two = sref[b] == z

        @pl.when(two)
        def _(t=t, a=a, z=z, b=b, rows=rows):
          mask = riota < b
          s1 = jnp.sum(jnp.where(mask, rows, 0.0), axis=0, keepdims=True)
          s2 = jnp.sum(jnp.where(mask, 0.0, rows), axis=0, keepdims=True)
          acc3[t, pl.ds(a, 1), :] += s1
          acc3[t, pl.ds(z, 1), :] += s2
          cnt3[t, pl.ds(a, 1), 0:1] += b.astype(jnp.float32)
          cnt3[t, pl.ds(z, 1), 0:1] += jnp.float32(R) - b.astype(jnp.float32)

        @pl.when(jnp.logical_not(two))
        def _(t=t, href=href, sref=sref):
          def rloop(r, carry):
            sidx = sref[r]
            acc3[t, pl.ds(sidx, 1), :] += href[pl.ds(r, 1), :]
            cnt3[t, pl.ds(sidx, 1), 0:1] += 1.0
            return carry
          lax.fori_loop(0, R, rloop, 0)

    @pl.when(i == nb - 1)
    def _():
      acc_out[...] = acc3[...]
      cnt_out[...] = cnt3[...]

  grid = (nb,)
  return pl.pallas_call(
      body,
      grid=grid,
      in_specs=[
          pl.BlockSpec((R,), lambda i: (i,), memory_space=pltpu.SMEM),
          pl.BlockSpec((R,), lambda i: (i,), memory_space=pltpu.SMEM),
          pl.BlockSpec((R,), lambda i: (i,), memory_space=pltpu.SMEM),
          pl.BlockSpec((R, D), lambda i: (i, 0)),
          pl.BlockSpec((R, D), lambda i: (i, 0)),
          pl.BlockSpec((R, D), lambda i: (i, 0)),
      ],
      out_specs=[
          pl.BlockSpec((NT, NG, D), lambda i: (0, 0, 0)),
          pl.BlockSpec((NT, NG, CW), lambda i: (0, 0, 0)),
      ],
      out_shape=[
          jax.ShapeDtypeStruct((NT, NG, D), jnp.float32),
          jax.ShapeDtypeStruct((NT, NG, CW), jnp.float32),
      ],
      scratch_shapes=[
          pltpu.VMEM((NT, NG, D), jnp.float32),
          pltpu.VMEM((NT, NG, CW), jnp.float32),
      ],
  )


def _attention_tc(acc, cnt, acc_t, cnt_t, W1, b1, W2):
  def body(acc_ref, cnt_ref, acct_ref, cntt_ref, W1_ref, b1_ref, W2_ref,
           out_ref):
    w1 = W1_ref[...]
    b1v = b1_ref[...]
    w2 = W2_ref[...]
    zs, ss = [], []
    for t in range(NT):
      a = acc_ref[0, t] + acc_ref[1, t] + acct_ref[t]
      cT = jnp.sum(cnt_ref[0, t] + cnt_ref[1, t] + cntt_ref[t],
                   axis=1, keepdims=True)
      z = a / jnp.maximum(cT, 1.0)
      zs.append(z)
      hzs = jnp.tanh(jnp.dot(z, w1, preferred_element_type=jnp.float32)
                     + b1v[None, :])
      ss.append(jnp.dot(hzs, w2, preferred_element_type=jnp.float32))
    sstack = jnp.concatenate(ss, axis=1)
    m = jnp.max(sstack, axis=1, keepdims=True)
    e = jnp.exp(sstack - m)
    beta = e / jnp.sum(e, axis=1, keepdims=True)
    out = beta[:, 0:1] * zs[0] + beta[:, 1:2] * zs[1] + beta[:, 2:3] * zs[2]
    out_ref[...] = out

  return pl.pallas_call(
      body,
      out_shape=jax.ShapeDtypeStruct((NG, D), jnp.float32),
  )(acc, cnt, acc_t, cnt_t, W1, b1, W2)


TC_BLOCKS = 390


def kernel(h0, h1, h2, seg0, seg1, seg2, W1, b1, W2, b2):
  n = h0.shape[0]
  n_tc = min(TC_BLOCKS * R, (n // R) * R)
  n_sc = n - n_tc
  ones = jnp.zeros((R, CW), jnp.float32).at[:, 0].set(1.0)
  zacc = jnp.zeros((NG, D), jnp.float32)
  zcnt = jnp.zeros((NG, CW), jnp.float32)
  s0 = seg0.astype(jnp.int32)
  s1 = seg1.astype(jnp.int32)
  s2 = seg2.astype(jnp.int32)
  sc = _sc_segment_sums(n_sc)
  acc, cnt = sc(h0[n_tc:], s0[n_tc:], h1[n_tc:], s1[n_tc:],
                h2[n_tc:], s2[n_tc:], ones, zacc, zcnt)
  acc_t, cnt_t = _tc_segment_sums(n_tc)(
      s0[:n_tc], s1[:n_tc], s2[:n_tc],
      h0[:n_tc], h1[:n_tc], h2[:n_tc])
  return _attention_tc(acc, cnt, acc_t, cnt_t, W1, b1, W2)


# speedup vs baseline: 1.8561x; 1.8561x over previous
import functools

import jax
import jax.numpy as jnp
from jax import lax
from jax.experimental import pallas as pl
from jax.experimental.pallas import tpu as pltpu
from jax.experimental.pallas import tpu_sc as plsc

NG = 256
D = 128
NT = 3
R = 128
CW = 128


def _sc_segment_sums(n):
  info = plsc.get_sparse_core_info()
  nc, ns = info.num_cores, info.num_subcores
  nw = nc * ns
  nfull = n // R
  rem = n - nfull * R
  rows_per_tile = NG // ns

  mesh = plsc.VectorSubcoreMesh(core_axis_name="c", subcore_axis_name="s")

  out_type = [
      jax.ShapeDtypeStruct((nc, NT, NG, D), jnp.float32),
      jax.ShapeDtypeStruct((nc, NT, NG, CW), jnp.float32),
  ]
  scratch = [
      pltpu.VMEM((R,), jnp.int32),
      pltpu.VMEM((R,), jnp.int32),
      pltpu.VMEM((R, D), jnp.float32),
      pltpu.VMEM((R, D), jnp.float32),
      pltpu.VMEM((R, CW), jnp.float32),
      pltpu.VMEM((rem, ), jnp.int32) if rem else pltpu.VMEM((8,), jnp.int32),
      pltpu.VMEM((max(rem, 1), D), jnp.float32),
      pltpu.SemaphoreType.DMA,
      pltpu.SemaphoreType.DMA,
      pltpu.SemaphoreType.DMA,
      pltpu.SemaphoreType.DMA,
      pltpu.VMEM((16, D), jnp.float32),
      pltpu.VMEM((16, D), jnp.float32),
      pltpu.VMEM((16, CW), jnp.float32),
      pltpu.VMEM((16, D), jnp.float32),
      pltpu.VMEM((16, D), jnp.float32),
      pltpu.VMEM((16, CW), jnp.float32),
      pltpu.VMEM((16, CW), jnp.float32),
      pltpu.SMEM((2,), jnp.int32),
  ] + [pltpu.VMEM_SHARED((NG, D), jnp.float32) for _ in range(NT)] \
    + [pltpu.VMEM_SHARED((NG, CW), jnp.float32) for _ in range(NT)]

  @functools.partial(pl.kernel, mesh=mesh, out_type=out_type,
                     scratch_types=scratch)
  def k(h0, s0, h1, s1, h2, s2, ones_hbm, zacc_hbm, zcnt_hbm,
        acc_out, cnt_out,
        seg_v0, seg_v1, rows_v0, rows_v1, ones_v, segr_v, rowsr_v,
        sem0, sem1, ssem0, ssem1, sums_v0, sums_v1, cntr_v,
        sums2_v0, sums2_v1, cnt2_v0, cnt2_v1, path_sm,
        acc0_sh, acc1_sh, acc2_sh, cnt0_sh, cnt1_sh, cnt2_sh):
    accs = (acc0_sh, acc1_sh, acc2_sh)
    cnts = (cnt0_sh, cnt1_sh, cnt2_sh)
    bufs = ((seg_v0, rows_v0, sem0), (seg_v1, rows_v1, sem1))
    ssems = (ssem0, ssem1)
    sumsb = (sums_v0, sums_v1)
    sums2b = (sums2_v0, sums2_v1)
    cnt2b = (cnt2_v0, cnt2_v1)
    c = lax.axis_index("c")
    s = lax.axis_index("s")
    w = s * nc + c

    zsl = pl.ds(s * rows_per_tile, rows_per_tile)
    for t in range(NT):
      pltpu.sync_copy(zacc_hbm.at[zsl], accs[t].at[zsl])
      pltpu.sync_copy(zcnt_hbm.at[zsl], cnts[t].at[zsl])
    pltpu.sync_copy(ones_hbm, ones_v)
    for buf in (sums_v0, sums_v1, sums2_v0, sums2_v1):
      pltpu.sync_copy(zacc_hbm.at[pl.ds(0, 16)], buf)
    for buf in (cntr_v, cnt2_v0, cnt2_v1):
      pltpu.sync_copy(zcnt_hbm.at[pl.ds(0, 16)], buf)
    path_sm[0] = 0
    path_sm[1] = 0
    for j in range(CW // 16):
      cntr_v[0, pl.ds(16 * j, 16)] = jnp.full((16,), 1.0, jnp.float32)
    plsc.subcore_barrier()

    nk = (nfull - w + nw - 1) // nw

    def issue(i, segb, rowsb, sem, seg, h):
      base = (w + i * nw) * R
      pltpu.async_copy(seg.at[pl.ds(base, R)], segb, sem)
      pltpu.async_copy(h.at[pl.ds(base, R)], rowsb, sem)

    def drain(segb, rowsb, sem, seg, h):
      pltpu.make_async_copy(seg.at[pl.ds(0, R)], segb, sem).wait()
      pltpu.make_async_copy(h.at[pl.ds(0, R)], rowsb, sem).wait()

    def wait_small(q):
      pltpu.make_async_copy(zacc_hbm.at[pl.ds(0, 16)], sumsb[q],
                            ssems[q]).wait()
      pltpu.make_async_copy(zcnt_hbm.at[pl.ds(0, 16)], cntr_v,
                            ssems[q]).wait()

    def wait_big(q, h):
      pltpu.make_async_copy(h.at[pl.ds(0, R)], bufs[q][1], ssems[q]).wait()
      pltpu.make_async_copy(ones_hbm, ones_v, ssems[q]).wait()

    def drain_outstanding(q, h):
      pq = path_sm[q]
      @pl.when(pq == 1)
      def _():
        wait_small(q)
      @pl.when(pq == 2)
      def _():
        wait_big(q, h)
      path_sm[q] = 0

    for t, (h, seg) in enumerate(((h0, s0), (h1, s1), (h2, s2))):
      issue(0, *bufs[0], seg, h)

      def body(i, carry, h=h, seg=seg, t=t):
        for p in range(2):
          @pl.when(lax.rem(i, 2) == p)
          def _(p=p):
            segb, rowsb, sem = bufs[p]
            sums_v = sumsb[p]
            sums2_v = sums2b[p]
            cnt2_v = cnt2b[p]
            ssem = ssems[p]
            drain(segb, rowsb, sem, seg, h)
            drain_outstanding(1 - p, h)
            @pl.when(i + 1 < nk)
            def _():
              issue(i + 1, *bufs[1 - p], seg, h)
            v0 = segb[pl.ds(0, 16)]
            vlast = segb[pl.ds(R - 16, 16)]
            first = v0[0]
            last = vlast[15]
            uni = first == last

            @pl.when(uni)
            def _():
              def sbody(r, acc):
                out = []
                for j in range(D // 16):
                  a = acc[j]
                  for u in range(8):
                    a = a + rowsb[8 * r + u, pl.ds(16 * j, 16)]
                  out.append(a)
                return tuple(out)
              acc = lax.fori_loop(
                  0, R // 8, sbody,
                  tuple(jnp.zeros((16,), jnp.float32)
                        for _ in range(D // 16)))
              for j in range(D // 16):
                sums_v[0, pl.ds(16 * j, 16)] = acc[j]
              pltpu.async_copy(sums_v, accs[t].at[v0], ssem, add=True)
              pltpu.async_copy(cntr_v, cnts[t].at[v0], ssem, add=True)
              path_sm[p] = 1

            @pl.when(jnp.logical_not(uni))
            def _():
              svs = [segb[pl.ds(16 * g, 16)] for g in range(R // 16)]
              e0 = [sv[0] for sv in svs]
              e15 = [sv[15] for sv in svs]
              m = [a != z for a, z in zip(e0, e15)]
              nnu = jnp.int32(0)
              gstar = jnp.int32(0)
              unif_ok = jnp.bool_(True)
              for g in range(R // 16):
                nnu = nnu + m[g].astype(jnp.int32)
                gstar = gstar + jnp.logical_and(
                    e0[g] == first, e15[g] == first).astype(jnp.int32)
                unif_ok = jnp.logical_and(
                    unif_ok,
                    m[g] | (e0[g] == first) | (e0[g] == last))
              svb = svs[-1]
              for g in range(R // 16 - 2, -1, -1):
                svb = jnp.where(m[g], svs[g], svb)
              minv = svs[0]
              maxv = svs[0]
              for g in range(1, R // 16):
                minv = jnp.minimum(minv, svs[g])
                maxv = jnp.maximum(maxv, svs[g])
              nfv = jnp.zeros((16,), jnp.float32)
              nlv = jnp.zeros((16,), jnp.float32)
              for g in range(R // 16):
                nfv = nfv + jnp.where(svs[g] == minv, 1.0, 0.0)
                nlv = nlv + jnp.where(svs[g] == maxv, 1.0, 0.0)
              b_within = jnp.int32(0)
              inset = jnp.bool_(True)
              for kk in range(16):
                ev = svb[kk]
                b_within = b_within + (ev == first).astype(jnp.int32)
                inset = jnp.logical_and(inset, (ev == first) | (ev == last))
              b = 16 * gstar + b_within
              two = ((nnu <= 1) & inset & unif_ok
                     & jnp.logical_not(m[0]) & jnp.logical_not(m[-1]))

              @pl.when(two)
              def _():
                def rbody(r, acc):
                  return tuple(acc[j] + rowsb[r, pl.ds(16 * j, 16)]
                               for j in range(D // 16))
                z8 = tuple(jnp.zeros((16,), jnp.float32)
                           for _ in range(D // 16))
                s1 = lax.fori_loop(0, b, rbody, z8)
                s2 = lax.fori_loop(b, R, rbody, z8)
                for j in range(D // 16):
                  sums2_v[0, pl.ds(16 * j, 16)] = s1[j]
                  sums2_v[1, pl.ds(16 * j, 16)] = s2[j]
                cnt2_v[0, pl.ds(0, 16)] = nfv
                cnt2_v[1, pl.ds(0, 16)] = nlv
                lane1 = lax.iota(jnp.int32, 16) == 1
                idx2 = jnp.where(lane1, maxv, minv)
                pltpu.async_copy(sums2_v, accs[t].at[idx2], ssem, add=True)
                pltpu.async_copy(cnt2_v, cnts[t].at[idx2], ssem, add=True)
                path_sm[p] = 1

              @pl.when(jnp.logical_not(two))
              def _():
                pltpu.async_copy(rowsb, accs[t].at[segb], ssem, add=True)
                pltpu.async_copy(ones_v, cnts[t].at[segb], ssem, add=True)
                path_sm[p] = 2
        return carry
      lax.fori_loop(0, nk, body, 0)
      drain_outstanding(0, h)
      drain_outstanding(1, h)

    if rem:
      @pl.when(w == nw - 1)
      def _():
        for t, (h, seg) in enumerate(((h0, s0), (h1, s1), (h2, s2))):
          pltpu.sync_copy(seg.at[pl.ds(nfull * R, rem)], segr_v)
          pltpu.sync_copy(h.at[pl.ds(nfull * R, rem)], rowsr_v)
          pltpu.sync_copy(rowsr_v, accs[t].at[segr_v], add=True)
          pltpu.sync_copy(ones_v.at[pl.ds(0, rem)], cnts[t].at[segr_v],
                          add=True)

    plsc.subcore_barrier()

    for t in range(NT):
      pltpu.sync_copy(accs[t].at[zsl], acc_out.at[c, t, zsl])
      pltpu.sync_copy(cnts[t].at[zsl], cnt_out.at[c, t, zsl])

  return k


SB = 1024


def _tc_segment_sums(n_tc):
  nb = n_tc // SB

  def body(s0_ref, s1_ref, s2_ref, h0_ref, h1_ref, h2_ref,
           acc_out, cnt_out, acc3, cnt3):
    i = pl.program_id(0)

    @pl.when(i == 0)
    def _():
      acc3[...] = jnp.zeros_like(acc3)
      cnt3[...] = jnp.zeros_like(cnt3)

    iota_seg = lax.broadcasted_iota(jnp.int32, (NG, SB), 0)
    for t, (sref, href) in enumerate(
        ((s0_ref, h0_ref), (s1_ref, h1_ref), (s2_ref, h2_ref))):
      oh = (iota_seg == sref[0]).astype(jnp.float32)
      acc3[t] += jnp.dot(oh, href[...],
                         preferred_element_type=jnp.float32)
      cnt3[t, :, 0:1] += jnp.sum(oh, axis=1, keepdims=True)

    @pl.when(i == nb - 1)
    def _():
      acc_out[...] = acc3[...]
      cnt_out[...] = cnt3[...]

  return pl.pallas_call(
      body,
      grid=(nb,),
      in_specs=[
          pl.BlockSpec((1, 1, SB), lambda i: (i, 0, 0)),
          pl.BlockSpec((1, 1, SB), lambda i: (i, 0, 0)),
          pl.BlockSpec((1, 1, SB), lambda i: (i, 0, 0)),
          pl.BlockSpec((SB, D), lambda i: (i, 0)),
          pl.BlockSpec((SB, D), lambda i: (i, 0)),
          pl.BlockSpec((SB, D), lambda i: (i, 0)),
      ],
      out_specs=[
          pl.BlockSpec((NT, NG, D), lambda i: (0, 0, 0)),
          pl.BlockSpec((NT, NG, CW), lambda i: (0, 0, 0)),
      ],
      out_shape=[
          jax.ShapeDtypeStruct((NT, NG, D), jnp.float32),
          jax.ShapeDtypeStruct((NT, NG, CW), jnp.float32),
      ],
      scratch_shapes=[
          pltpu.VMEM((NT, NG, D), jnp.float32),
          pltpu.VMEM((NT, NG, CW), jnp.float32),
      ],
  )


def _attention_tc(acc, cnt, acc_t, cnt_t, W1, b1, W2):
  def body(acc_ref, cnt_ref, acct_ref, cntt_ref, W1_ref, b1_ref, W2_ref,
           out_ref):
    w1 = W1_ref[...]
    b1v = b1_ref[...]
    w2 = W2_ref[...]
    zs, ss = [], []
    for t in range(NT):
      a = acc_ref[0, t] + acc_ref[1, t] + acct_ref[t]
      cT = jnp.sum(cnt_ref[0, t] + cnt_ref[1, t] + cntt_ref[t],
                   axis=1, keepdims=True)
      z = a / jnp.maximum(cT, 1.0)
      zs.append(z)
      hzs = jnp.tanh(jnp.dot(z, w1, preferred_element_type=jnp.float32)
                     + b1v[None, :])
      ss.append(jnp.dot(hzs, w2, preferred_element_type=jnp.float32))
    sstack = jnp.concatenate(ss, axis=1)
    m = jnp.max(sstack, axis=1, keepdims=True)
    e = jnp.exp(sstack - m)
    beta = e / jnp.sum(e, axis=1, keepdims=True)
    out = beta[:, 0:1] * zs[0] + beta[:, 1:2] * zs[1] + beta[:, 2:3] * zs[2]
    out_ref[...] = out

  return pl.pallas_call(
      body,
      out_shape=jax.ShapeDtypeStruct((NG, D), jnp.float32),
  )(acc, cnt, acc_t, cnt_t, W1, b1, W2)


TC_SBLOCKS = 48


def kernel(h0, h1, h2, seg0, seg1, seg2, W1, b1, W2, b2):
  n = h0.shape[0]
  n_tc = min(TC_SBLOCKS * SB, (n // SB) * SB)
  n_sc = n - n_tc
  ones = jnp.zeros((R, CW), jnp.float32).at[:, 0].set(1.0)
  zacc = jnp.zeros((NG, D), jnp.float32)
  zcnt = jnp.zeros((NG, CW), jnp.float32)
  s0 = seg0.astype(jnp.int32)
  s1 = seg1.astype(jnp.int32)
  s2 = seg2.astype(jnp.int32)
  sc = _sc_segment_sums(n_sc)
  acc, cnt = sc(h0[n_tc:], s0[n_tc:], h1[n_tc:], s1[n_tc:],
                h2[n_tc:], s2[n_tc:], ones, zacc, zcnt)
  nbt = n_tc // SB
  acc_t, cnt_t = _tc_segment_sums(n_tc)(
      s0[:n_tc].reshape(nbt, 1, SB), s1[:n_tc].reshape(nbt, 1, SB),
      s2[:n_tc].reshape(nbt, 1, SB),
      h0[:n_tc], h1[:n_tc], h2[:n_tc])
  return _attention_tc(acc, cnt, acc_t, cnt_t, W1, b1, W2)


# speedup vs baseline: 3.3189x; 1.7881x over previous
import functools

import jax
import jax.numpy as jnp
from jax import lax
from jax.experimental import pallas as pl
from jax.experimental.pallas import tpu as pltpu
from jax.experimental.pallas import tpu_sc as plsc

NG = 256
D = 128
NT = 3
R = 128
CW = 128


def _sc_segment_sums(n, off):
  info = plsc.get_sparse_core_info()
  nc, ns = info.num_cores, info.num_subcores
  nw = nc * ns
  nfull = (n - off) // R
  rem = n - off - nfull * R
  rows_per_tile = NG // ns

  mesh = plsc.VectorSubcoreMesh(core_axis_name="c", subcore_axis_name="s")

  out_type = [
      jax.ShapeDtypeStruct((nc, NT, NG, D), jnp.float32),
      jax.ShapeDtypeStruct((nc, NT, NG, CW), jnp.float32),
  ]
  scratch = [
      pltpu.VMEM((R,), jnp.int32),
      pltpu.VMEM((R,), jnp.int32),
      pltpu.VMEM((R, D), jnp.float32),
      pltpu.VMEM((R, D), jnp.float32),
      pltpu.VMEM((R, CW), jnp.float32),
      pltpu.VMEM((rem, ), jnp.int32) if rem else pltpu.VMEM((8,), jnp.int32),
      pltpu.VMEM((max(rem, 1), D), jnp.float32),
      pltpu.SemaphoreType.DMA,
      pltpu.SemaphoreType.DMA,
      pltpu.SemaphoreType.DMA,
      pltpu.SemaphoreType.DMA,
      pltpu.VMEM((16, D), jnp.float32),
      pltpu.VMEM((16, D), jnp.float32),
      pltpu.VMEM((16, CW), jnp.float32),
      pltpu.VMEM((16, D), jnp.float32),
      pltpu.VMEM((16, D), jnp.float32),
      pltpu.VMEM((16, CW), jnp.float32),
      pltpu.VMEM((16, CW), jnp.float32),
      pltpu.SMEM((2,), jnp.int32),
  ] + [pltpu.VMEM_SHARED((NG, D), jnp.float32) for _ in range(NT)] \
    + [pltpu.VMEM_SHARED((NG, CW), jnp.float32) for _ in range(NT)]

  @functools.partial(pl.kernel, mesh=mesh, out_type=out_type,
                     scratch_types=scratch)
  def k(h0, s0, h1, s1, h2, s2, ones_hbm, zacc_hbm, zcnt_hbm,
        acc_out, cnt_out,
        seg_v0, seg_v1, rows_v0, rows_v1, ones_v, segr_v, rowsr_v,
        sem0, sem1, ssem0, ssem1, sums_v0, sums_v1, cntr_v,
        sums2_v0, sums2_v1, cnt2_v0, cnt2_v1, path_sm,
        acc0_sh, acc1_sh, acc2_sh, cnt0_sh, cnt1_sh, cnt2_sh):
    accs = (acc0_sh, acc1_sh, acc2_sh)
    cnts = (cnt0_sh, cnt1_sh, cnt2_sh)
    bufs = ((seg_v0, rows_v0, sem0), (seg_v1, rows_v1, sem1))
    ssems = (ssem0, ssem1)
    sumsb = (sums_v0, sums_v1)
    sums2b = (sums2_v0, sums2_v1)
    cnt2b = (cnt2_v0, cnt2_v1)
    c = lax.axis_index("c")
    s = lax.axis_index("s")
    w = s * nc + c

    zsl = pl.ds(s * rows_per_tile, rows_per_tile)
    for t in range(NT):
      pltpu.sync_copy(zacc_hbm.at[zsl], accs[t].at[zsl])
      pltpu.sync_copy(zcnt_hbm.at[zsl], cnts[t].at[zsl])
    pltpu.sync_copy(ones_hbm, ones_v)
    for buf in (sums_v0, sums_v1, sums2_v0, sums2_v1):
      pltpu.sync_copy(zacc_hbm.at[pl.ds(0, 16)], buf)
    for buf in (cntr_v, cnt2_v0, cnt2_v1):
      pltpu.sync_copy(zcnt_hbm.at[pl.ds(0, 16)], buf)
    path_sm[0] = 0
    path_sm[1] = 0
    for j in range(CW // 16):
      cntr_v[0, pl.ds(16 * j, 16)] = jnp.full((16,), 1.0, jnp.float32)
    plsc.subcore_barrier()

    nk = (nfull - w + nw - 1) // nw

    def issue(i, segb, rowsb, sem, seg, h):
      base = off + (w + i * nw) * R
      pltpu.async_copy(seg.at[pl.ds(base, R)], segb, sem)
      pltpu.async_copy(h.at[pl.ds(base, R)], rowsb, sem)

    def drain(segb, rowsb, sem, seg, h):
      pltpu.make_async_copy(seg.at[pl.ds(0, R)], segb, sem).wait()
      pltpu.make_async_copy(h.at[pl.ds(0, R)], rowsb, sem).wait()

    def wait_small(q):
      pltpu.make_async_copy(zacc_hbm.at[pl.ds(0, 16)], sumsb[q],
                            ssems[q]).wait()
      pltpu.make_async_copy(zcnt_hbm.at[pl.ds(0, 16)], cntr_v,
                            ssems[q]).wait()

    def wait_big(q, h):
      pltpu.make_async_copy(h.at[pl.ds(0, R)], bufs[q][1], ssems[q]).wait()
      pltpu.make_async_copy(ones_hbm, ones_v, ssems[q]).wait()

    def drain_outstanding(q, h):
      pq = path_sm[q]
      @pl.when(pq == 1)
      def _():
        wait_small(q)
      @pl.when(pq == 2)
      def _():
        wait_big(q, h)
      path_sm[q] = 0

    for t, (h, seg) in enumerate(((h0, s0), (h1, s1), (h2, s2))):
      issue(0, *bufs[0], seg, h)

      def body(i, carry, h=h, seg=seg, t=t):
        for p in range(2):
          @pl.when(lax.rem(i, 2) == p)
          def _(p=p):
            segb, rowsb, sem = bufs[p]
            sums_v = sumsb[p]
            sums2_v = sums2b[p]
            cnt2_v = cnt2b[p]
            ssem = ssems[p]
            drain(segb, rowsb, sem, seg, h)
            drain_outstanding(1 - p, h)
            @pl.when(i + 1 < nk)
            def _():
              issue(i + 1, *bufs[1 - p], seg, h)
            v0 = segb[pl.ds(0, 16)]
            vlast = segb[pl.ds(R - 16, 16)]
            first = v0[0]
            last = vlast[15]
            uni = first == last

            @pl.when(uni)
            def _():
              def sbody(r, acc):
                out = []
                for j in range(D // 16):
                  a = acc[j]
                  for u in range(8):
                    a = a + rowsb[8 * r + u, pl.ds(16 * j, 16)]
                  out.append(a)
                return tuple(out)
              acc = lax.fori_loop(
                  0, R // 8, sbody,
                  tuple(jnp.zeros((16,), jnp.float32)
                        for _ in range(D // 16)))
              for j in range(D // 16):
                sums_v[0, pl.ds(16 * j, 16)] = acc[j]
              pltpu.async_copy(sums_v, accs[t].at[v0], ssem, add=True)
              pltpu.async_copy(cntr_v, cnts[t].at[v0], ssem, add=True)
              path_sm[p] = 1

            @pl.when(jnp.logical_not(uni))
            def _():
              svs = [segb[pl.ds(16 * g, 16)] for g in range(R // 16)]
              e0 = [sv[0] for sv in svs]
              e15 = [sv[15] for sv in svs]
              m = [a != z for a, z in zip(e0, e15)]
              nnu = jnp.int32(0)
              gstar = jnp.int32(0)
              unif_ok = jnp.bool_(True)
              for g in range(R // 16):
                nnu = nnu + m[g].astype(jnp.int32)
                gstar = gstar + jnp.logical_and(
                    e0[g] == first, e15[g] == first).astype(jnp.int32)
                unif_ok = jnp.logical_and(
                    unif_ok,
                    m[g] | (e0[g] == first) | (e0[g] == last))
              svb = svs[-1]
              for g in range(R // 16 - 2, -1, -1):
                svb = jnp.where(m[g], svs[g], svb)
              minv = svs[0]
              maxv = svs[0]
              for g in range(1, R // 16):
                minv = jnp.minimum(minv, svs[g])
                maxv = jnp.maximum(maxv, svs[g])
              nfv = jnp.zeros((16,), jnp.float32)
              nlv = jnp.zeros((16,), jnp.float32)
              for g in range(R // 16):
                nfv = nfv + jnp.where(svs[g] == minv, 1.0, 0.0)
                nlv = nlv + jnp.where(svs[g] == maxv, 1.0, 0.0)
              b_within = jnp.int32(0)
              inset = jnp.bool_(True)
              for kk in range(16):
                ev = svb[kk]
                b_within = b_within + (ev == first).astype(jnp.int32)
                inset = jnp.logical_and(inset, (ev == first) | (ev == last))
              b = 16 * gstar + b_within
              two = ((nnu <= 1) & inset & unif_ok
                     & jnp.logical_not(m[0]) & jnp.logical_not(m[-1]))

              @pl.when(two)
              def _():
                def rbody(r, acc):
                  return tuple(acc[j] + rowsb[r, pl.ds(16 * j, 16)]
                               for j in range(D // 16))
                z8 = tuple(jnp.zeros((16,), jnp.float32)
                           for _ in range(D // 16))
                s1 = lax.fori_loop(0, b, rbody, z8)
                s2 = lax.fori_loop(b, R, rbody, z8)
                for j in range(D // 16):
                  sums2_v[0, pl.ds(16 * j, 16)] = s1[j]
                  sums2_v[1, pl.ds(16 * j, 16)] = s2[j]
                cnt2_v[0, pl.ds(0, 16)] = nfv
                cnt2_v[1, pl.ds(0, 16)] = nlv
                lane1 = lax.iota(jnp.int32, 16) == 1
                idx2 = jnp.where(lane1, maxv, minv)
                pltpu.async_copy(sums2_v, accs[t].at[idx2], ssem, add=True)
                pltpu.async_copy(cnt2_v, cnts[t].at[idx2], ssem, add=True)
                path_sm[p] = 1

              @pl.when(jnp.logical_not(two))
              def _():
                pltpu.async_copy(rowsb, accs[t].at[segb], ssem, add=True)
                pltpu.async_copy(ones_v, cnts[t].at[segb], ssem, add=True)
                path_sm[p] = 2
        return carry
      lax.fori_loop(0, nk, body, 0)
      drain_outstanding(0, h)
      drain_outstanding(1, h)

    if rem:
      @pl.when(w == nw - 1)
      def _():
        for t, (h, seg) in enumerate(((h0, s0), (h1, s1), (h2, s2))):
          pltpu.sync_copy(seg.at[pl.ds(off + nfull * R, rem)], segr_v)
          pltpu.sync_copy(h.at[pl.ds(off + nfull * R, rem)], rowsr_v)
          pltpu.sync_copy(rowsr_v, accs[t].at[segr_v], add=True)
          pltpu.sync_copy(ones_v.at[pl.ds(0, rem)], cnts[t].at[segr_v],
                          add=True)

    plsc.subcore_barrier()

    for t in range(NT):
      pltpu.sync_copy(accs[t].at[zsl], acc_out.at[c, t, zsl])
      pltpu.sync_copy(cnts[t].at[zsl], cnt_out.at[c, t, zsl])

  return k


SB = 1024


def _tc_segment_sums(n_tc):
  nb = n_tc // SB

  def body(s0_ref, s1_ref, s2_ref, h0_ref, h1_ref, h2_ref,
           acc_out, cnt_out, acc3, cnt3):
    i = pl.program_id(0)

    @pl.when(i == 0)
    def _():
      acc3[...] = jnp.zeros_like(acc3)
      cnt3[...] = jnp.zeros_like(cnt3)

    iota_seg = lax.broadcasted_iota(jnp.int32, (NG, SB), 0)
    for t, (sref, href) in enumerate(
        ((s0_ref, h0_ref), (s1_ref, h1_ref), (s2_ref, h2_ref))):
      oh = (iota_seg == sref[0]).astype(jnp.float32)
      acc3[t] += jnp.dot(oh, href[...],
                         preferred_element_type=jnp.float32)
      cnt3[t, :, 0:1] += jnp.sum(oh, axis=1, keepdims=True)

    @pl.when(i == nb - 1)
    def _():
      acc_out[...] = acc3[...]
      cnt_out[...] = cnt3[...]

  return pl.pallas_call(
      body,
      grid=(nb,),
      in_specs=[
          pl.BlockSpec((1, 1, SB), lambda i: (i, 0, 0)),
          pl.BlockSpec((1, 1, SB), lambda i: (i, 0, 0)),
          pl.BlockSpec((1, 1, SB), lambda i: (i, 0, 0)),
          pl.BlockSpec((SB, D), lambda i: (i, 0)),
          pl.BlockSpec((SB, D), lambda i: (i, 0)),
          pl.BlockSpec((SB, D), lambda i: (i, 0)),
      ],
      out_specs=[
          pl.BlockSpec((NT, NG, D), lambda i: (0, 0, 0)),
          pl.BlockSpec((NT, NG, CW), lambda i: (0, 0, 0)),
      ],
      out_shape=[
          jax.ShapeDtypeStruct((NT, NG, D), jnp.float32),
          jax.ShapeDtypeStruct((NT, NG, CW), jnp.float32),
      ],
      scratch_shapes=[
          pltpu.VMEM((NT, NG, D), jnp.float32),
          pltpu.VMEM((NT, NG, CW), jnp.float32),
      ],
  )


def _attention_tc(acc, cnt, acc_t, cnt_t, W1, b1, W2):
  def body(acc_ref, cnt_ref, acct_ref, cntt_ref, W1_ref, b1_ref, W2_ref,
           out_ref):
    w1 = W1_ref[...]
    b1v = b1_ref[...]
    w2 = W2_ref[...]
    zs, ss = [], []
    for t in range(NT):
      a = acc_ref[0, t] + acc_ref[1, t] + acct_ref[t]
      cT = jnp.sum(cnt_ref[0, t] + cnt_ref[1, t] + cntt_ref[t],
                   axis=1, keepdims=True)
      z = a / jnp.maximum(cT, 1.0)
      zs.append(z)
      hzs = jnp.tanh(jnp.dot(z, w1, preferred_element_type=jnp.float32)
                     + b1v[None, :])
      ss.append(jnp.dot(hzs, w2, preferred_element_type=jnp.float32))
    sstack = jnp.concatenate(ss, axis=1)
    m = jnp.max(sstack, axis=1, keepdims=True)
    e = jnp.exp(sstack - m)
    beta = e / jnp.sum(e, axis=1, keepdims=True)
    out = beta[:, 0:1] * zs[0] + beta[:, 1:2] * zs[1] + beta[:, 2:3] * zs[2]
    out_ref[...] = out

  return pl.pallas_call(
      body,
      out_shape=jax.ShapeDtypeStruct((NG, D), jnp.float32),
  )(acc, cnt, acc_t, cnt_t, W1, b1, W2)


TC_SBLOCKS = 48


def kernel(h0, h1, h2, seg0, seg1, seg2, W1, b1, W2, b2):
  n = h0.shape[0]
  n_tc = min(TC_SBLOCKS * SB, (n // SB) * SB)
  n_sc = n - n_tc
  ones = jnp.zeros((R, CW), jnp.float32).at[:, 0].set(1.0)
  zacc = jnp.zeros((NG, D), jnp.float32)
  zcnt = jnp.zeros((NG, CW), jnp.float32)
  s0 = seg0.astype(jnp.int32)
  s1 = seg1.astype(jnp.int32)
  s2 = seg2.astype(jnp.int32)
  nbt = n_tc // SB
  sc = _sc_segment_sums(n, n_tc)
  acc, cnt = sc(h0, s0, h1, s1, h2, s2, ones, zacc, zcnt)
  acc_t, cnt_t = _tc_segment_sums(n_tc)(
      s0[:n_tc].reshape(nbt, 1, SB), s1[:n_tc].reshape(nbt, 1, SB),
      s2[:n_tc].reshape(nbt, 1, SB),
      h0, h1, h2)
  return _attention_tc(acc, cnt, acc_t, cnt_t, W1, b1, W2)


# speedup vs baseline: 3.6381x; 1.0962x over previous
import functools

import jax
import jax.numpy as jnp
from jax import lax
from jax.experimental import pallas as pl
from jax.experimental.pallas import tpu as pltpu
from jax.experimental.pallas import tpu_sc as plsc

NG = 256
D = 128
NT = 3
R = 128
CW = 128


def _sc_segment_sums(n, off):
  info = plsc.get_sparse_core_info()
  nc, ns = info.num_cores, info.num_subcores
  nw = nc * ns
  nfull = (n - off) // R
  rem = n - off - nfull * R
  rows_per_tile = NG // ns

  mesh = plsc.VectorSubcoreMesh(core_axis_name="c", subcore_axis_name="s")

  out_type = [
      jax.ShapeDtypeStruct((nc, NT, NG, D), jnp.float32),
      jax.ShapeDtypeStruct((nc, NT, NG, CW), jnp.float32),
  ]
  scratch = [
      pltpu.VMEM((R,), jnp.int32),
      pltpu.VMEM((R,), jnp.int32),
      pltpu.VMEM((R, D), jnp.float32),
      pltpu.VMEM((R, D), jnp.float32),
      pltpu.VMEM((R, CW), jnp.float32),
      pltpu.VMEM((rem, ), jnp.int32) if rem else pltpu.VMEM((8,), jnp.int32),
      pltpu.VMEM((max(rem, 1), D), jnp.float32),
      pltpu.SemaphoreType.DMA,
      pltpu.SemaphoreType.DMA,
      pltpu.SemaphoreType.DMA,
      pltpu.SemaphoreType.DMA,
      pltpu.VMEM((16, D), jnp.float32),
      pltpu.VMEM((16, D), jnp.float32),
      pltpu.VMEM((16, CW), jnp.float32),
      pltpu.VMEM((16, D), jnp.float32),
      pltpu.VMEM((16, D), jnp.float32),
      pltpu.VMEM((16, CW), jnp.float32),
      pltpu.VMEM((16, CW), jnp.float32),
      pltpu.SMEM((2,), jnp.int32),
  ] + [pltpu.VMEM_SHARED((NG, D), jnp.float32) for _ in range(NT)] \
    + [pltpu.VMEM_SHARED((NG, CW), jnp.float32) for _ in range(NT)]

  @functools.partial(pl.kernel, mesh=mesh, out_type=out_type,
                     scratch_types=scratch)
  def k(h0, s0, h1, s1, h2, s2, ones_hbm, zacc_hbm, zcnt_hbm,
        acc_out, cnt_out,
        seg_v0, seg_v1, rows_v0, rows_v1, ones_v, segr_v, rowsr_v,
        sem0, sem1, ssem0, ssem1, sums_v0, sums_v1, cntr_v,
        sums2_v0, sums2_v1, cnt2_v0, cnt2_v1, path_sm,
        acc0_sh, acc1_sh, acc2_sh, cnt0_sh, cnt1_sh, cnt2_sh):
    accs = (acc0_sh, acc1_sh, acc2_sh)
    cnts = (cnt0_sh, cnt1_sh, cnt2_sh)
    bufs = ((seg_v0, rows_v0, sem0), (seg_v1, rows_v1, sem1))
    ssems = (ssem0, ssem1)
    sumsb = (sums_v0, sums_v1)
    sums2b = (sums2_v0, sums2_v1)
    cnt2b = (cnt2_v0, cnt2_v1)
    c = lax.axis_index("c")
    s = lax.axis_index("s")
    w = s * nc + c

    zsl = pl.ds(s * rows_per_tile, rows_per_tile)
    for t in range(NT):
      pltpu.sync_copy(zacc_hbm.at[zsl], accs[t].at[zsl])
      pltpu.sync_copy(zcnt_hbm.at[zsl], cnts[t].at[zsl])
    pltpu.sync_copy(ones_hbm, ones_v)
    for buf in (sums_v0, sums_v1, sums2_v0, sums2_v1):
      pltpu.sync_copy(zacc_hbm.at[pl.ds(0, 16)], buf)
    for buf in (cntr_v, cnt2_v0, cnt2_v1):
      pltpu.sync_copy(zcnt_hbm.at[pl.ds(0, 16)], buf)
    path_sm[0] = 0
    path_sm[1] = 0
    for j in range(CW // 16):
      cntr_v[0, pl.ds(16 * j, 16)] = jnp.full((16,), 1.0, jnp.float32)
    plsc.subcore_barrier()

    nk = (nfull - w + nw - 1) // nw

    def issue(i, segb, rowsb, sem, seg, h):
      base = off + (w + i * nw) * R
      pltpu.async_copy(seg.at[pl.ds(base, R)], segb, sem)
      pltpu.async_copy(h.at[pl.ds(base, R)], rowsb, sem)

    def drain(segb, rowsb, sem, seg, h):
      pltpu.make_async_copy(seg.at[pl.ds(0, R)], segb, sem).wait()
      pltpu.make_async_copy(h.at[pl.ds(0, R)], rowsb, sem).wait()

    def wait_small(q):
      pltpu.make_async_copy(zacc_hbm.at[pl.ds(0, 16)], sumsb[q],
                            ssems[q]).wait()
      pltpu.make_async_copy(zcnt_hbm.at[pl.ds(0, 16)], cntr_v,
                            ssems[q]).wait()

    def wait_big(q, h):
      pltpu.make_async_copy(h.at[pl.ds(0, R)], bufs[q][1], ssems[q]).wait()
      pltpu.make_async_copy(ones_hbm, ones_v, ssems[q]).wait()

    def drain_outstanding(q, h):
      pq = path_sm[q]
      @pl.when(pq == 1)
      def _():
        wait_small(q)
      @pl.when(pq == 2)
      def _():
        wait_big(q, h)
      path_sm[q] = 0

    for t, (h, seg) in enumerate(((h0, s0), (h1, s1), (h2, s2))):
      issue(0, *bufs[0], seg, h)

      def body(i, carry, h=h, seg=seg, t=t):
        for p in range(2):
          @pl.when(lax.rem(i, 2) == p)
          def _(p=p):
            segb, rowsb, sem = bufs[p]
            sums_v = sumsb[p]
            sums2_v = sums2b[p]
            cnt2_v = cnt2b[p]
            ssem = ssems[p]
            drain(segb, rowsb, sem, seg, h)
            drain_outstanding(1 - p, h)
            @pl.when(i + 1 < nk)
            def _():
              issue(i + 1, *bufs[1 - p], seg, h)
            v0 = segb[pl.ds(0, 16)]
            vlast = segb[pl.ds(R - 16, 16)]
            first = v0[0]
            last = vlast[15]
            uni = first == last

            @pl.when(uni)
            def _():
              def sbody(r, acc):
                out = []
                for j in range(D // 16):
                  a = acc[j]
                  for u in range(8):
                    a = a + rowsb[8 * r + u, pl.ds(16 * j, 16)]
                  out.append(a)
                return tuple(out)
              acc = lax.fori_loop(
                  0, R // 8, sbody,
                  tuple(jnp.zeros((16,), jnp.float32)
                        for _ in range(D // 16)))
              for j in range(D // 16):
                sums_v[0, pl.ds(16 * j, 16)] = acc[j]
              pltpu.async_copy(sums_v, accs[t].at[v0], ssem, add=True)
              pltpu.async_copy(cntr_v, cnts[t].at[v0], ssem, add=True)
              path_sm[p] = 1

            @pl.when(jnp.logical_not(uni))
            def _():
              svs = [segb[pl.ds(16 * g, 16)] for g in range(R // 16)]
              e0 = [sv[0] for sv in svs]
              e15 = [sv[15] for sv in svs]
              m = [a != z for a, z in zip(e0, e15)]
              nnu = jnp.int32(0)
              gstar = jnp.int32(0)
              unif_ok = jnp.bool_(True)
              for g in range(R // 16):
                nnu = nnu + m[g].astype(jnp.int32)
                gstar = gstar + jnp.logical_and(
                    e0[g] == first, e15[g] == first).astype(jnp.int32)
                unif_ok = jnp.logical_and(
                    unif_ok,
                    m[g] | (e0[g] == first) | (e0[g] == last))
              svb = svs[-1]
              for g in range(R // 16 - 2, -1, -1):
                svb = jnp.where(m[g], svs[g], svb)
              minv = svs[0]
              maxv = svs[0]
              for g in range(1, R // 16):
                minv = jnp.minimum(minv, svs[g])
                maxv = jnp.maximum(maxv, svs[g])
              nfv = jnp.zeros((16,), jnp.float32)
              nlv = jnp.zeros((16,), jnp.float32)
              for g in range(R // 16):
                nfv = nfv + jnp.where(svs[g] == minv, 1.0, 0.0)
                nlv = nlv + jnp.where(svs[g] == maxv, 1.0, 0.0)
              b_within = jnp.int32(0)
              inset = jnp.bool_(True)
              for kk in range(16):
                ev = svb[kk]
                b_within = b_within + (ev == first).astype(jnp.int32)
                inset = jnp.logical_and(inset, (ev == first) | (ev == last))
              b = 16 * gstar + b_within
              two = ((nnu <= 1) & inset & unif_ok
                     & jnp.logical_not(m[0]) & jnp.logical_not(m[-1]))

              @pl.when(two)
              def _():
                def rbody(r, acc):
                  return tuple(acc[j] + rowsb[r, pl.ds(16 * j, 16)]
                               for j in range(D // 16))
                z8 = tuple(jnp.zeros((16,), jnp.float32)
                           for _ in range(D // 16))
                s1 = lax.fori_loop(0, b, rbody, z8)
                s2 = lax.fori_loop(b, R, rbody, z8)
                for j in range(D // 16):
                  sums2_v[0, pl.ds(16 * j, 16)] = s1[j]
                  sums2_v[1, pl.ds(16 * j, 16)] = s2[j]
                cnt2_v[0, pl.ds(0, 16)] = nfv
                cnt2_v[1, pl.ds(0, 16)] = nlv
                lane1 = lax.iota(jnp.int32, 16) == 1
                idx2 = jnp.where(lane1, maxv, minv)
                pltpu.async_copy(sums2_v, accs[t].at[idx2], ssem, add=True)
                pltpu.async_copy(cnt2_v, cnts[t].at[idx2], ssem, add=True)
                path_sm[p] = 1

              @pl.when(jnp.logical_not(two))
              def _():
                pltpu.async_copy(rowsb, accs[t].at[segb], ssem, add=True)
                pltpu.async_copy(ones_v, cnts[t].at[segb], ssem, add=True)
                path_sm[p] = 2
        return carry
      lax.fori_loop(0, nk, body, 0)
      drain_outstanding(0, h)
      drain_outstanding(1, h)

    if rem:
      @pl.when(w == nw - 1)
      def _():
        for t, (h, seg) in enumerate(((h0, s0), (h1, s1), (h2, s2))):
          pltpu.sync_copy(seg.at[pl.ds(off + nfull * R, rem)], segr_v)
          pltpu.sync_copy(h.at[pl.ds(off + nfull * R, rem)], rowsr_v)
          pltpu.sync_copy(rowsr_v, accs[t].at[segr_v], add=True)
          pltpu.sync_copy(ones_v.at[pl.ds(0, rem)], cnts[t].at[segr_v],
                          add=True)

    plsc.subcore_barrier()

    for t in range(NT):
      pltpu.sync_copy(accs[t].at[zsl], acc_out.at[c, t, zsl])
      pltpu.sync_copy(cnts[t].at[zsl], cnt_out.at[c, t, zsl])

  return k


SB = 1024


def _tc_segment_sums(n_tc):
  nb = n_tc // SB

  def body(s0_ref, s1_ref, s2_ref, h0_ref, h1_ref, h2_ref,
           acc_out, cnt_out, acc3, cnt3):
    i = pl.program_id(0)

    @pl.when(i == 0)
    def _():
      acc3[...] = jnp.zeros_like(acc3)
      cnt3[...] = jnp.zeros_like(cnt3)

    iota_seg = lax.broadcasted_iota(jnp.int32, (NG, SB), 0)
    for t, (sref, href) in enumerate(
        ((s0_ref, h0_ref), (s1_ref, h1_ref), (s2_ref, h2_ref))):
      oh = (iota_seg == sref[0]).astype(jnp.float32)
      acc3[t] += jnp.dot(oh, href[...],
                         preferred_element_type=jnp.float32)
      cnt3[t, :, 0:1] += jnp.sum(oh, axis=1, keepdims=True)

    @pl.when(i == nb - 1)
    def _():
      acc_out[...] = acc3[...]
      cnt_out[...] = cnt3[...]

  return pl.pallas_call(
      body,
      grid=(nb,),
      in_specs=[
          pl.BlockSpec((1, 1, SB), lambda i: (i, 0, 0)),
          pl.BlockSpec((1, 1, SB), lambda i: (i, 0, 0)),
          pl.BlockSpec((1, 1, SB), lambda i: (i, 0, 0)),
          pl.BlockSpec((SB, D), lambda i: (i, 0)),
          pl.BlockSpec((SB, D), lambda i: (i, 0)),
          pl.BlockSpec((SB, D), lambda i: (i, 0)),
      ],
      out_specs=[
          pl.BlockSpec((NT, NG, D), lambda i: (0, 0, 0)),
          pl.BlockSpec((NT, NG, CW), lambda i: (0, 0, 0)),
      ],
      out_shape=[
          jax.ShapeDtypeStruct((NT, NG, D), jnp.float32),
          jax.ShapeDtypeStruct((NT, NG, CW), jnp.float32),
      ],
      scratch_shapes=[
          pltpu.VMEM((NT, NG, D), jnp.float32),
          pltpu.VMEM((NT, NG, CW), jnp.float32),
      ],
  )


def _attention_tc(acc, cnt, acc_t, cnt_t, W1, b1, W2):
  def body(acc_ref, cnt_ref, acct_ref, cntt_ref, W1_ref, b1_ref, W2_ref,
           out_ref):
    w1 = W1_ref[...]
    b1v = b1_ref[...]
    w2 = W2_ref[...]
    zs, ss = [], []
    for t in range(NT):
      a = acc_ref[0, t] + acc_ref[1, t] + acct_ref[t]
      cT = jnp.sum(cnt_ref[0, t] + cnt_ref[1, t] + cntt_ref[t],
                   axis=1, keepdims=True)
      z = a / jnp.maximum(cT, 1.0)
      zs.append(z)
      hzs = jnp.tanh(jnp.dot(z, w1, preferred_element_type=jnp.float32)
                     + b1v[None, :])
      ss.append(jnp.dot(hzs, w2, preferred_element_type=jnp.float32))
    sstack = jnp.concatenate(ss, axis=1)
    m = jnp.max(sstack, axis=1, keepdims=True)
    e = jnp.exp(sstack - m)
    beta = e / jnp.sum(e, axis=1, keepdims=True)
    out = beta[:, 0:1] * zs[0] + beta[:, 1:2] * zs[1] + beta[:, 2:3] * zs[2]
    out_ref[...] = out

  return pl.pallas_call(
      body,
      out_shape=jax.ShapeDtypeStruct((NG, D), jnp.float32),
  )(acc, cnt, acc_t, cnt_t, W1, b1, W2)


TC_SBLOCKS = 62


def kernel(h0, h1, h2, seg0, seg1, seg2, W1, b1, W2, b2):
  n = h0.shape[0]
  n_tc = min(TC_SBLOCKS * SB, (n // SB) * SB)
  n_sc = n - n_tc
  ones = jnp.zeros((R, CW), jnp.float32).at[:, 0].set(1.0)
  zacc = jnp.zeros((NG, D), jnp.float32)
  zcnt = jnp.zeros((NG, CW), jnp.float32)
  s0 = seg0.astype(jnp.int32)
  s1 = seg1.astype(jnp.int32)
  s2 = seg2.astype(jnp.int32)
  nbt = n_tc // SB
  sc = _sc_segment_sums(n, n_tc)
  acc, cnt = sc(h0, s0, h1, s1, h2, s2, ones, zacc, zcnt)
  acc_t, cnt_t = _tc_segment_sums(n_tc)(
      s0[:n_tc].reshape(nbt, 1, SB), s1[:n_tc].reshape(nbt, 1, SB),
      s2[:n_tc].reshape(nbt, 1, SB),
      h0, h1, h2)
  return _attention_tc(acc, cnt, acc_t, cnt_t, W1, b1, W2)
